# Initial kernel scaffold; baseline (speedup 1.0000x reference)
import jax
import jax.numpy as jnp
from jax.experimental import pallas as pl


def kernel(edge1, edge2, feat1, feat2, W_gcn, b_gcn, fc1_W, fc1_b, fc2_W, fc2_b):
    raise NotImplementedError("write your pallas kernel here")


# speedup vs baseline: 2.9589x; 2.9589x over previous
import functools

import jax
import jax.numpy as jnp
from jax import lax
from jax.experimental import pallas as pl
from jax.experimental.pallas import tpu as pltpu

NN = 10000
DD = 128
NPAD = 10240
BI = 512
BJ = 512
NIB = NPAD // BI
NJB = NPAD // BJ
INV_TEMP = 2.0


def _k1_body(x_ref, w_ref, deg_ref, hs_ref, dinv_ref):
    dinv = lax.rsqrt(jnp.maximum(deg_ref[...], 1.0))
    h = jnp.dot(x_ref[...], w_ref[...], preferred_element_type=jnp.float32)
    hs_ref[...] = h * dinv
    dinv_ref[...] = dinv


def _k1(x, w, deg):
    B = 2000
    grid = (NN // B,)
    return pl.pallas_call(
        _k1_body,
        grid=grid,
        in_specs=[
            pl.BlockSpec((B, DD), lambda i: (i, 0)),
            pl.BlockSpec((DD, DD), lambda i: (0, 0)),
            pl.BlockSpec((B, 1), lambda i: (i, 0)),
        ],
        out_specs=[
            pl.BlockSpec((B, DD), lambda i: (i, 0)),
            pl.BlockSpec((B, 1), lambda i: (i, 0)),
        ],
        out_shape=[
            jax.ShapeDtypeStruct((NN, DD), jnp.float32),
            jax.ShapeDtypeStruct((NN, 1), jnp.float32),
        ],
    )(x, w, deg)


def _k5_body(acc_ref, hs_ref, dinv_ref, b_ref, w1_ref, b1_ref, w2_ref, b2_ref,
             a_ref):
    g = dinv_ref[...] * (acc_ref[...] + hs_ref[...]) + b_ref[...]
    t = jnp.dot(g, w1_ref[...], preferred_element_type=jnp.float32) + b1_ref[...]
    z = jnp.where(t > 0.0, t, jnp.exp(jnp.minimum(t, 0.0)) - 1.0)
    z2 = jnp.dot(z, w2_ref[...], preferred_element_type=jnp.float32) + b2_ref[...]
    nrm = jnp.sqrt(jnp.sum(z2 * z2, axis=1, keepdims=True))
    a_ref[...] = z2 / jnp.maximum(nrm, 1e-12)


def _k5(acc, hs, dinv, b, w1, b1, w2, b2):
    B = 2000
    grid = (NN // B,)
    row = lambda i: (i, 0)
    full = lambda i: (0, 0)
    return pl.pallas_call(
        _k5_body,
        grid=grid,
        in_specs=[
            pl.BlockSpec((B, DD), row),
            pl.BlockSpec((B, DD), row),
            pl.BlockSpec((B, 1), row),
            pl.BlockSpec((1, DD), full),
            pl.BlockSpec((DD, DD), full),
            pl.BlockSpec((1, DD), full),
            pl.BlockSpec((DD, DD), full),
            pl.BlockSpec((1, DD), full),
        ],
        out_specs=pl.BlockSpec((B, DD), row),
        out_shape=jax.ShapeDtypeStruct((NN, DD), jnp.float32),
    )(acc, hs, dinv, b, w1, b1, w2, b2)


def _dott(x, y):
    return lax.dot_general(x, y, (((1,), (1,)), ((), ())),
                           preferred_element_type=jnp.float32)


def _k6_body(aI_ref, bI_ref, aJ_ref, bJ_ref,
             r11_ref, r22_ref, r12_ref, c12_ref, d11_ref, d22_ref, d12_ref):
    i = pl.program_id(0)
    j = pl.program_id(1)
    aI = aI_ref[...]
    bI = bI_ref[...]
    aJ = aJ_ref[...]
    bJ = bJ_ref[...]
    s11 = _dott(aI, aJ)
    s22 = _dott(bI, bJ)
    s12 = _dott(aI, bJ)
    jm = ((lax.broadcasted_iota(jnp.int32, (1, BJ), 1) + j * BJ) < NN
          ).astype(jnp.float32)
    im = ((lax.broadcasted_iota(jnp.int32, (BI, 1), 0) + i * BI) < NN
          ).astype(jnp.float32)
    e11 = jnp.exp(s11 * INV_TEMP) * jm
    e22 = jnp.exp(s22 * INV_TEMP) * jm
    e12 = jnp.exp(s12 * INV_TEMP) * jm

    @pl.when(jnp.logical_and(i == 0, j == 0))
    def _():
        r11_ref[...] = jnp.zeros_like(r11_ref)
        r22_ref[...] = jnp.zeros_like(r22_ref)
        r12_ref[...] = jnp.zeros_like(r12_ref)
        c12_ref[...] = jnp.zeros_like(c12_ref)
        d11_ref[...] = jnp.zeros_like(d11_ref)
        d22_ref[...] = jnp.zeros_like(d22_ref)
        d12_ref[...] = jnp.zeros_like(d12_ref)

    ri = pl.ds(i, 1)
    r11_ref[ri, :] += jnp.sum(e11, axis=1).reshape(1, BI)
    r22_ref[ri, :] += jnp.sum(e22, axis=1).reshape(1, BI)
    r12_ref[ri, :] += jnp.sum(e12, axis=1).reshape(1, BI)
    c12_ref[pl.ds(j, 1), :] += jnp.sum(e12 * im, axis=0, keepdims=True)

    @pl.when(i == j)
    def _():
        eye = (lax.broadcasted_iota(jnp.int32, (BI, BJ), 0)
               == lax.broadcasted_iota(jnp.int32, (BI, BJ), 1)
               ).astype(jnp.float32)
        d11_ref[ri, :] += jnp.sum(e11 * eye, axis=1).reshape(1, BI)
        d22_ref[ri, :] += jnp.sum(e22 * eye, axis=1).reshape(1, BI)
        d12_ref[ri, :] += jnp.sum(s12 * eye, axis=1).reshape(1, BI)


def _k6(a1p, a2p):
    accspec = pl.BlockSpec((NIB, BI), lambda i, j: (0, 0))
    acc = jax.ShapeDtypeStruct((NIB, BI), jnp.float32)
    return pl.pallas_call(
        _k6_body,
        grid=(NIB, NJB),
        in_specs=[
            pl.BlockSpec((BI, DD), lambda i, j: (i, 0)),
            pl.BlockSpec((BI, DD), lambda i, j: (i, 0)),
            pl.BlockSpec((BJ, DD), lambda i, j: (j, 0)),
            pl.BlockSpec((BJ, DD), lambda i, j: (j, 0)),
        ],
        out_specs=[accspec] * 7,
        out_shape=[acc] * 7,
    )(a1p, a1p, a2p, a2p)


def _k7_body(r11_ref, r22_ref, r12_ref, c12_ref, d11_ref, d22_ref, d12_ref,
             mask_ref, out_ref):
    x1 = r11_ref[...] + r12_ref[...] - d11_ref[...]
    x2 = r22_ref[...] + c12_ref[...] - d22_ref[...]
    num = INV_TEMP * d12_ref[...]
    l = 0.5 * ((jnp.log(x1) - num) + (jnp.log(x2) - num))
    out_ref[0, 0] = jnp.sum(l * mask_ref[...]) * (1.0 / NN)


def _k7(r11, r22, r12, c12p, d11, d22, d12, mask):
    return pl.pallas_call(
        _k7_body,
        out_specs=pl.BlockSpec(memory_space=pltpu.SMEM),
        out_shape=jax.ShapeDtypeStruct((1, 1), jnp.float32),
    )(r11, r22, r12, c12p, d11, d22, d12, mask)


def _deg_jnp(edge):
    return jnp.zeros((NN,), jnp.float32).at[edge[1]].add(1.0) + 1.0


def _scatter_jnp(hs, edge):
    return jnp.zeros((NN, DD), jnp.float32).at[edge[1]].add(hs[edge[0]])


def kernel(edge1, edge2, feat1, feat2, W_gcn, b_gcn, fc1_W, fc1_b, fc2_W,
           fc2_b):
    deg1 = _deg_jnp(edge1).reshape(NN, 1)
    deg2 = _deg_jnp(edge2).reshape(NN, 1)
    hs1, dinv1 = _k1(feat1, W_gcn, deg1)
    hs2, dinv2 = _k1(feat2, W_gcn, deg2)
    acc1 = _scatter_jnp(hs1, edge1)
    acc2 = _scatter_jnp(hs2, edge2)
    b2d = b_gcn.reshape(1, DD)
    b1d = fc1_b.reshape(1, DD)
    b2d2 = fc2_b.reshape(1, DD)
    a1 = _k5(acc1, hs1, dinv1, b2d, fc1_W, b1d, fc2_W, b2d2)
    a2 = _k5(acc2, hs2, dinv2, b2d, fc1_W, b1d, fc2_W, b2d2)
    a1p = jnp.pad(a1, ((0, NPAD - NN), (0, 0)))
    a2p = jnp.pad(a2, ((0, NPAD - NN), (0, 0)))
    r11, r22, r12, c12p, d11, d22, d12 = _k6(a1p, a2p)
    mask = (jnp.arange(NPAD) < NN).astype(jnp.float32).reshape(NIB, BI)
    loss = _k7(r11, r22, r12, c12p, d11, d22, d12, mask)
    return loss[0, 0]


# speedup vs baseline: 6.7366x; 2.2767x over previous
import functools

import jax
import jax.numpy as jnp
from jax import lax
from jax.experimental import pallas as pl
from jax.experimental.pallas import tpu as pltpu
from jax.experimental.pallas import tpu_sc as plsc

NN = 10000
DD = 128
EE = 160000
SC_NC = 2
SC_NS = 16
SC_NW = SC_NC * SC_NS
ECH = 128
NCHUNK = EE // ECH
KMAX = -(-NCHUNK // SC_NW)
ACC_ROWS = 10240
STRIPE = ACC_ROWS // SC_NS
NPAD = 10240
BI = 512
BJ = 512
NIB = NPAD // BI
NJB = NPAD // BJ
INV_TEMP = 2.0


def _dinv_block(degp_ref, i, B):
    deg = jnp.sum(degp_ref[:, pl.ds(i * B, B)], axis=0) + 1.0
    return lax.rsqrt(deg)[:, None]


def _k1_body(x_ref, w_ref, degp_ref, hs_ref):
    i = pl.program_id(0)
    dinv = _dinv_block(degp_ref, i, x_ref.shape[0])
    h = jnp.dot(x_ref[...], w_ref[...], preferred_element_type=jnp.float32)
    hs_ref[...] = h * dinv


def _k1(x, w, degp):
    B = 2048
    grid = (NPAD // B,)
    return pl.pallas_call(
        _k1_body,
        grid=grid,
        in_specs=[
            pl.BlockSpec((B, DD), lambda i: (i, 0)),
            pl.BlockSpec((DD, DD), lambda i: (0, 0)),
            pl.BlockSpec((SC_NW, NPAD), lambda i: (0, 0)),
        ],
        out_specs=pl.BlockSpec((B, DD), lambda i: (i, 0)),
        out_shape=jax.ShapeDtypeStruct((NPAD, DD), jnp.float32),
    )(x, w, degp)


def _k5_body(acc_ref, accb_ref, hs_ref, degp_ref, b_ref, w1_ref, b1_ref,
             w2_ref, b2_ref, a_ref):
    i = pl.program_id(0)
    dinv = _dinv_block(degp_ref, i, acc_ref.shape[0])
    g = dinv * (acc_ref[...] + accb_ref[...] + hs_ref[...]) + b_ref[...]
    t = jnp.dot(g, w1_ref[...], preferred_element_type=jnp.float32) + b1_ref[...]
    z = jnp.where(t > 0.0, t, jnp.exp(jnp.minimum(t, 0.0)) - 1.0)
    z2 = jnp.dot(z, w2_ref[...], preferred_element_type=jnp.float32) + b2_ref[...]
    nrm = jnp.sqrt(jnp.sum(z2 * z2, axis=1, keepdims=True))
    a_ref[...] = z2 / jnp.maximum(nrm, 1e-12)


def _k5(acc, accb, hs, degp, b, w1, b1, w2, b2):
    B = 2048
    grid = (NPAD // B,)
    row = lambda i: (i, 0)
    full = lambda i: (0, 0)
    return pl.pallas_call(
        _k5_body,
        grid=grid,
        in_specs=[
            pl.BlockSpec((B, DD), row),
            pl.BlockSpec((B, DD), row),
            pl.BlockSpec((B, DD), row),
            pl.BlockSpec((SC_NW, NPAD), full),
            pl.BlockSpec((1, DD), full),
            pl.BlockSpec((DD, DD), full),
            pl.BlockSpec((1, DD), full),
            pl.BlockSpec((DD, DD), full),
            pl.BlockSpec((1, DD), full),
        ],
        out_specs=pl.BlockSpec((B, DD), row),
        out_shape=jax.ShapeDtypeStruct((NPAD, DD), jnp.float32),
    )(acc, accb, hs, degp, b, w1, b1, w2, b2)


def _dott(x, y):
    return lax.dot_general(x, y, (((1,), (1,)), ((), ())),
                           preferred_element_type=jnp.float32)


def _k6_body(aI_ref, bI_ref, aJ_ref, bJ_ref,
             r11_ref, r22_ref, r12_ref, c12_ref, d11_ref, d22_ref, d12_ref):
    i = pl.program_id(0)
    j = pl.program_id(1)
    aI = aI_ref[...]
    bI = bI_ref[...]
    aJ = aJ_ref[...]
    bJ = bJ_ref[...]
    s11 = _dott(aI, aJ)
    s22 = _dott(bI, bJ)
    s12 = _dott(aI, bJ)
    jm = ((lax.broadcasted_iota(jnp.int32, (1, BJ), 1) + j * BJ) < NN
          ).astype(jnp.float32)
    im = ((lax.broadcasted_iota(jnp.int32, (BI, 1), 0) + i * BI) < NN
          ).astype(jnp.float32)
    e11 = jnp.exp(s11 * INV_TEMP) * jm
    e22 = jnp.exp(s22 * INV_TEMP) * jm
    e12 = jnp.exp(s12 * INV_TEMP) * jm

    @pl.when(jnp.logical_and(i == 0, j == 0))
    def _():
        r11_ref[...] = jnp.zeros_like(r11_ref)
        r22_ref[...] = jnp.zeros_like(r22_ref)
        r12_ref[...] = jnp.zeros_like(r12_ref)
        c12_ref[...] = jnp.zeros_like(c12_ref)
        d11_ref[...] = jnp.zeros_like(d11_ref)
        d22_ref[...] = jnp.zeros_like(d22_ref)
        d12_ref[...] = jnp.zeros_like(d12_ref)

    ri = pl.ds(i, 1)
    r11_ref[ri, :] += jnp.sum(e11, axis=1).reshape(1, BI)
    r22_ref[ri, :] += jnp.sum(e22, axis=1).reshape(1, BI)
    r12_ref[ri, :] += jnp.sum(e12, axis=1).reshape(1, BI)
    c12_ref[pl.ds(j, 1), :] += jnp.sum(e12 * im, axis=0, keepdims=True)

    @pl.when(i == j)
    def _():
        eye = (lax.broadcasted_iota(jnp.int32, (BI, BJ), 0)
               == lax.broadcasted_iota(jnp.int32, (BI, BJ), 1)
               ).astype(jnp.float32)
        d11_ref[ri, :] += jnp.sum(e11 * eye, axis=1).reshape(1, BI)
        d22_ref[ri, :] += jnp.sum(e22 * eye, axis=1).reshape(1, BI)
        d12_ref[ri, :] += jnp.sum(s12 * eye, axis=1).reshape(1, BI)


def _k6(a1p, a2p):
    accspec = pl.BlockSpec((NIB, BI), lambda i, j: (0, 0))
    acc = jax.ShapeDtypeStruct((NIB, BI), jnp.float32)
    return pl.pallas_call(
        _k6_body,
        grid=(NIB, NJB),
        in_specs=[
            pl.BlockSpec((BI, DD), lambda i, j: (i, 0)),
            pl.BlockSpec((BI, DD), lambda i, j: (i, 0)),
            pl.BlockSpec((BJ, DD), lambda i, j: (j, 0)),
            pl.BlockSpec((BJ, DD), lambda i, j: (j, 0)),
        ],
        out_specs=[accspec] * 7,
        out_shape=[acc] * 7,
    )(a1p, a1p, a2p, a2p)


def _k7_body(r11_ref, r22_ref, r12_ref, c12_ref, d11_ref, d22_ref, d12_ref,
             mask_ref, out_ref):
    x1 = r11_ref[...] + r12_ref[...] - d11_ref[...]
    x2 = r22_ref[...] + c12_ref[...] - d22_ref[...]
    num = INV_TEMP * d12_ref[...]
    l = 0.5 * ((jnp.log(x1) - num) + (jnp.log(x2) - num))
    out_ref[0, 0] = jnp.sum(l * mask_ref[...]) * (1.0 / NN)


def _k7(r11, r22, r12, c12p, d11, d22, d12, mask):
    return pl.pallas_call(
        _k7_body,
        out_specs=pl.BlockSpec(memory_space=pltpu.SMEM),
        out_shape=jax.ShapeDtypeStruct((1, 1), jnp.float32),
    )(r11, r22, r12, c12p, d11, d22, d12, mask)


@functools.lru_cache(maxsize=None)
def _sc_mesh():
    return plsc.VectorSubcoreMesh(core_axis_name="c", subcore_axis_name="s",
                                  num_cores=SC_NC, num_subcores=SC_NS)


def _sc_deg(edge):
    return pl.kernel(
        _sc_deg_body,
        out_type=jax.ShapeDtypeStruct((SC_NW, NN), jnp.float32),
        mesh=_sc_mesh(),
        scratch_types=[
            pltpu.VMEM((ECH,), jnp.int32),
            pltpu.VMEM((ACC_ROWS,), jnp.float32),
        ],
        compiler_params=pltpu.CompilerParams(needs_layout_passes=False,
                                             use_tc_tiling_on_sc=False),
    )(edge)


def _sc_deg_body(edge_ref, deg_ref, idx_v, degloc):
    c = lax.axis_index("c")
    s = lax.axis_index("s")
    wid = s * SC_NC + c

    def zb(k, carry):
        degloc[pl.ds(k * 16, 16)] = jnp.zeros((16,), jnp.float32)
        return carry

    lax.fori_loop(0, ACC_ROWS // 16, zb, None)
    ones = jnp.ones((16,), jnp.float32)

    def chunk(k, carry):
        g = k * SC_NW + wid

        @pl.when(g < NCHUNK)
        def _():
            pltpu.sync_copy(edge_ref.at[1, pl.ds(g * ECH, ECH)], idx_v)
            for t in range(ECH // 16):
                idx = idx_v[pl.ds(t * 16, 16)]
                plsc.addupdate_scatter(degloc, [idx], ones)

        return carry

    lax.fori_loop(0, KMAX, chunk, None)
    pltpu.sync_copy(degloc.at[pl.ds(0, NN)], deg_ref.at[wid])


def _sc_scatter(hs, edge):
    return pl.kernel(
        _sc_scatter_body,
        out_type=jax.ShapeDtypeStruct((SC_NC, ACC_ROWS, DD), jnp.float32),
        mesh=_sc_mesh(),
        scratch_types=[
            pltpu.VMEM((ECH,), jnp.int32),
            pltpu.VMEM((ECH,), jnp.int32),
            pltpu.VMEM((ECH, DD), jnp.float32),
            pltpu.VMEM_SHARED((ACC_ROWS, DD), jnp.float32),
            pltpu.SemaphoreType.DMA,
        ],
        compiler_params=pltpu.CompilerParams(needs_layout_passes=False,
                                             use_tc_tiling_on_sc=False),
    )(hs, edge)


def _sc_scatter_body(hs_ref, edge_ref, acc_ref, src_v, dst_v, rows_v, acc_sh,
                     sem):
    c = lax.axis_index("c")
    s = lax.axis_index("s")
    wid = s * SC_NC + c

    def zb(k, carry):
        r = k // (DD // 16)
        t = k % (DD // 16)
        rows_v[r, pl.ds(t * 16, 16)] = jnp.zeros((16,), jnp.float32)
        return carry

    lax.fori_loop(0, ECH * (DD // 16), zb, None)
    for sblk in range(STRIPE // ECH):
        pltpu.sync_copy(rows_v, acc_sh.at[pl.ds(s * STRIPE + sblk * ECH, ECH)])
    plsc.subcore_barrier()

    def chunk(k, carry):
        g = k * SC_NW + wid

        @pl.when(g < NCHUNK)
        def _():
            base = g * ECH
            pltpu.sync_copy(edge_ref.at[0, pl.ds(base, ECH)], src_v)
            pltpu.sync_copy(edge_ref.at[1, pl.ds(base, ECH)], dst_v)
            pltpu.async_copy(hs_ref.at[src_v], rows_v, sem).wait()
            pltpu.sync_copy(rows_v, acc_sh.at[dst_v], add=True)

        return carry

    lax.fori_loop(0, KMAX, chunk, None)
    plsc.subcore_barrier()
    pltpu.sync_copy(acc_sh.at[pl.ds(s * STRIPE, STRIPE)],
                    acc_ref.at[c, pl.ds(s * STRIPE, STRIPE)])


def kernel(edge1, edge2, feat1, feat2, W_gcn, b_gcn, fc1_W, fc1_b, fc2_W,
           fc2_b):
    f1p = jnp.pad(feat1, ((0, NPAD - NN), (0, 0)))
    f2p = jnp.pad(feat2, ((0, NPAD - NN), (0, 0)))
    degp1 = jnp.pad(_sc_deg(edge1), ((0, 0), (0, NPAD - NN)))
    degp2 = jnp.pad(_sc_deg(edge2), ((0, 0), (0, NPAD - NN)))
    hs1 = _k1(f1p, W_gcn, degp1)
    hs2 = _k1(f2p, W_gcn, degp2)
    accp1 = _sc_scatter(hs1, edge1)
    accp2 = _sc_scatter(hs2, edge2)
    b2d = b_gcn.reshape(1, DD)
    b1d = fc1_b.reshape(1, DD)
    b2d2 = fc2_b.reshape(1, DD)
    a1p = _k5(accp1[0], accp1[1], hs1, degp1, b2d, fc1_W, b1d, fc2_W, b2d2)
    a2p = _k5(accp2[0], accp2[1], hs2, degp2, b2d, fc1_W, b1d, fc2_W, b2d2)
    r11, r22, r12, c12p, d11, d22, d12 = _k6(a1p, a2p)
    mask = (jnp.arange(NPAD) < NN).astype(jnp.float32).reshape(NIB, BI)
    loss = _k7(r11, r22, r12, c12p, d11, d22, d12, mask)
    return loss[0, 0]


# speedup vs baseline: 8.8876x; 1.3193x over previous
import functools

import jax
import jax.numpy as jnp
from jax import lax
from jax.experimental import pallas as pl
from jax.experimental.pallas import tpu as pltpu
from jax.experimental.pallas import tpu_sc as plsc

NN = 10000
DD = 128
EE = 160000
SC_NC = 2
SC_NS = 16
SC_NW = SC_NC * SC_NS
ECH = 128
NCHUNK = EE // ECH
KMAX = -(-NCHUNK // SC_NW)
ACC_ROWS = 10240
STRIPE = ACC_ROWS // SC_NS
NPAD = 10240
BI = 512
BJ = 512
NIB = NPAD // BI
NJB = NPAD // BJ
INV_TEMP = 2.0


def _dinv_block(degp_ref, i, B):
    deg = jnp.sum(degp_ref[:, pl.ds(i * B, B)], axis=0) + 1.0
    return lax.rsqrt(deg)[:, None]


def _k1_body(x_ref, w_ref, degp_ref, hs_ref):
    i = pl.program_id(0)
    dinv = _dinv_block(degp_ref, i, x_ref.shape[0])
    h = jnp.dot(x_ref[...], w_ref[...], preferred_element_type=jnp.float32)
    hs_ref[...] = h * dinv


def _k1(x, w, degp):
    B = 2048
    grid = (NPAD // B,)
    return pl.pallas_call(
        _k1_body,
        grid=grid,
        in_specs=[
            pl.BlockSpec((B, DD), lambda i: (i, 0)),
            pl.BlockSpec((DD, DD), lambda i: (0, 0)),
            pl.BlockSpec((SC_NW, NPAD), lambda i: (0, 0)),
        ],
        out_specs=pl.BlockSpec((B, DD), lambda i: (i, 0)),
        out_shape=jax.ShapeDtypeStruct((NPAD, DD), jnp.float32),
    )(x, w, degp)


def _k5_body(acc_ref, accb_ref, hs_ref, degp_ref, b_ref, w1_ref, b1_ref,
             w2_ref, b2_ref, a_ref):
    i = pl.program_id(0)
    dinv = _dinv_block(degp_ref, i, acc_ref.shape[0])
    g = dinv * (acc_ref[...] + accb_ref[...] + hs_ref[...]) + b_ref[...]
    t = jnp.dot(g, w1_ref[...], preferred_element_type=jnp.float32) + b1_ref[...]
    z = jnp.where(t > 0.0, t, jnp.exp(jnp.minimum(t, 0.0)) - 1.0)
    z2 = jnp.dot(z, w2_ref[...], preferred_element_type=jnp.float32) + b2_ref[...]
    nrm = jnp.sqrt(jnp.sum(z2 * z2, axis=1, keepdims=True))
    a_ref[...] = z2 / jnp.maximum(nrm, 1e-12)


def _k5(acc, accb, hs, degp, b, w1, b1, w2, b2):
    B = 2048
    grid = (NPAD // B,)
    row = lambda i: (i, 0)
    full = lambda i: (0, 0)
    return pl.pallas_call(
        _k5_body,
        grid=grid,
        in_specs=[
            pl.BlockSpec((B, DD), row),
            pl.BlockSpec((B, DD), row),
            pl.BlockSpec((B, DD), row),
            pl.BlockSpec((SC_NW, NPAD), full),
            pl.BlockSpec((1, DD), full),
            pl.BlockSpec((DD, DD), full),
            pl.BlockSpec((1, DD), full),
            pl.BlockSpec((DD, DD), full),
            pl.BlockSpec((1, DD), full),
        ],
        out_specs=pl.BlockSpec((B, DD), row),
        out_shape=jax.ShapeDtypeStruct((NPAD, DD), jnp.float32),
    )(acc, accb, hs, degp, b, w1, b1, w2, b2)


def _dott(x, y):
    return lax.dot_general(x, y, (((1,), (1,)), ((), ())),
                           preferred_element_type=jnp.float32)


def _k6_body(aI_ref, bI_ref, aJ_ref, bJ_ref,
             r11_ref, r12_ref, d11_ref, d12_ref,
             r22_ref, c12_ref, d22_ref):
    i = pl.program_id(0)
    j = pl.program_id(1)
    aI = aI_ref[...]
    bI = bI_ref[...]
    aJ = aJ_ref[...]
    bJ = bJ_ref[...]
    s11 = _dott(aI, aJ)
    s22t = _dott(bJ, bI)
    s12 = _dott(aI, bJ)
    e11 = jnp.exp(s11 * INV_TEMP)
    e22t = jnp.exp(s22t * INV_TEMP)
    e12 = jnp.exp(s12 * INV_TEMP)
    jm1 = ((lax.broadcasted_iota(jnp.int32, (1, BJ), 1) + j * BJ) < NN
           ).astype(jnp.float32)
    im1 = ((lax.broadcasted_iota(jnp.int32, (1, BI), 1) + i * BI) < NN
           ).astype(jnp.float32)

    @pl.when(j == 0)
    def _():
        r11_ref[...] = jnp.zeros_like(r11_ref)
        r12_ref[...] = jnp.zeros_like(r12_ref)
        d11_ref[...] = jnp.zeros_like(d11_ref)
        d12_ref[...] = jnp.zeros_like(d12_ref)

    @pl.when(jnp.logical_and(i == 0, j == 0))
    def _():
        r22_ref[...] = jnp.zeros_like(r22_ref)
        c12_ref[...] = jnp.zeros_like(c12_ref)
        d22_ref[...] = jnp.zeros_like(d22_ref)

    r11_ref[...] += _dott(e11, jm1)
    r12_ref[...] += _dott(e12, jm1)
    r22_ref[pl.ds(i, 1), :] += jnp.dot(jm1, e22t,
                                       preferred_element_type=jnp.float32)
    c12_ref[pl.ds(j, 1), :] += jnp.dot(im1, e12,
                                       preferred_element_type=jnp.float32)

    @pl.when(i == j)
    def _():
        eye = (lax.broadcasted_iota(jnp.int32, (BI, BJ), 0)
               == lax.broadcasted_iota(jnp.int32, (BI, BJ), 1)
               ).astype(jnp.float32)
        d11_ref[...] += _dott(e11 * eye, jm1)
        d12_ref[...] += _dott(s12 * eye, jm1)
        d22_ref[pl.ds(i, 1), :] += jnp.dot(jm1, e22t * eye,
                                           preferred_element_type=jnp.float32)


def _k6(a1p, a2p):
    colspec = pl.BlockSpec((BI, 1), lambda i, j: (i, 0))
    col = jax.ShapeDtypeStruct((NPAD, 1), jnp.float32)
    lanespec = pl.BlockSpec((NIB, BI), lambda i, j: (0, 0))
    lane = jax.ShapeDtypeStruct((NIB, BI), jnp.float32)
    return pl.pallas_call(
        _k6_body,
        grid=(NIB, NJB),
        in_specs=[
            pl.BlockSpec((BI, DD), lambda i, j: (i, 0)),
            pl.BlockSpec((BI, DD), lambda i, j: (i, 0)),
            pl.BlockSpec((BJ, DD), lambda i, j: (j, 0)),
            pl.BlockSpec((BJ, DD), lambda i, j: (j, 0)),
        ],
        out_specs=[colspec, colspec, colspec, colspec,
                   lanespec, lanespec, lanespec],
        out_shape=[col, col, col, col, lane, lane, lane],
    )(a1p, a1p, a2p, a2p)


def _k7_body(r11_ref, r12_ref, d11_ref, d12_ref, r22_ref, c12_ref, d22_ref,
             out_ref):
    k = pl.program_id(0)
    imc = ((lax.broadcasted_iota(jnp.int32, (BI, 1), 0) + k * BI) < NN
           ).astype(jnp.float32)
    x1 = r11_ref[...] + r12_ref[...] - d11_ref[...]
    p1 = jnp.sum((0.5 * jnp.log(x1) - INV_TEMP * d12_ref[...]) * imc)
    iml = ((lax.broadcasted_iota(jnp.int32, (1, BI), 1) + k * BI) < NN
           ).astype(jnp.float32)
    rk = pl.ds(k, 1)
    x2 = r22_ref[rk, :] + c12_ref[rk, :] - d22_ref[rk, :]
    p2 = jnp.sum(0.5 * jnp.log(x2) * iml)

    @pl.when(k == 0)
    def _():
        out_ref[0, 0] = 0.0

    out_ref[0, 0] += (p1 + p2) * (1.0 / NN)


def _k7(r11, r12, d11, d12, r22, c12, d22):
    colspec = pl.BlockSpec((BI, 1), lambda k: (k, 0))
    lanespec = pl.BlockSpec((NIB, BI), lambda k: (0, 0))
    return pl.pallas_call(
        _k7_body,
        grid=(NIB,),
        in_specs=[colspec, colspec, colspec, colspec,
                  lanespec, lanespec, lanespec],
        out_specs=pl.BlockSpec(memory_space=pltpu.SMEM),
        out_shape=jax.ShapeDtypeStruct((1, 1), jnp.float32),
    )(r11, r12, d11, d12, r22, c12, d22)


@functools.lru_cache(maxsize=None)
def _sc_mesh():
    return plsc.VectorSubcoreMesh(core_axis_name="c", subcore_axis_name="s",
                                  num_cores=SC_NC, num_subcores=SC_NS)


def _sc_deg(edge):
    return pl.kernel(
        _sc_deg_body,
        out_type=jax.ShapeDtypeStruct((SC_NW, NN), jnp.float32),
        mesh=_sc_mesh(),
        scratch_types=[
            pltpu.VMEM((ECH,), jnp.int32),
            pltpu.VMEM((ACC_ROWS,), jnp.float32),
        ],
        compiler_params=pltpu.CompilerParams(needs_layout_passes=False,
                                             use_tc_tiling_on_sc=False),
    )(edge)


def _sc_deg_body(edge_ref, deg_ref, idx_v, degloc):
    c = lax.axis_index("c")
    s = lax.axis_index("s")
    wid = s * SC_NC + c

    def zb(k, carry):
        degloc[pl.ds(k * 16, 16)] = jnp.zeros((16,), jnp.float32)
        return carry

    lax.fori_loop(0, ACC_ROWS // 16, zb, None)
    ones = jnp.ones((16,), jnp.float32)

    def chunk(k, carry):
        g = k * SC_NW + wid

        @pl.when(g < NCHUNK)
        def _():
            pltpu.sync_copy(edge_ref.at[1, pl.ds(g * ECH, ECH)], idx_v)
            for t in range(ECH // 16):
                idx = idx_v[pl.ds(t * 16, 16)]
                plsc.addupdate_scatter(degloc, [idx], ones)

        return carry

    lax.fori_loop(0, KMAX, chunk, None)
    pltpu.sync_copy(degloc.at[pl.ds(0, NN)], deg_ref.at[wid])


def _sc_scatter(hs, edge):
    return pl.kernel(
        _sc_scatter_body,
        out_type=jax.ShapeDtypeStruct((SC_NC, ACC_ROWS, DD), jnp.float32),
        mesh=_sc_mesh(),
        scratch_types=[
            pltpu.VMEM((ECH,), jnp.int32),
            pltpu.VMEM((ECH,), jnp.int32),
            pltpu.VMEM((ECH, DD), jnp.float32),
            pltpu.VMEM_SHARED((ACC_ROWS, DD), jnp.float32),
            pltpu.SemaphoreType.DMA,
        ],
        compiler_params=pltpu.CompilerParams(needs_layout_passes=False,
                                             use_tc_tiling_on_sc=False),
    )(hs, edge)


def _sc_scatter_body(hs_ref, edge_ref, acc_ref, src_v, dst_v, rows_v, acc_sh,
                     sem):
    c = lax.axis_index("c")
    s = lax.axis_index("s")
    wid = s * SC_NC + c

    def zb(k, carry):
        r = k // (DD // 16)
        t = k % (DD // 16)
        rows_v[r, pl.ds(t * 16, 16)] = jnp.zeros((16,), jnp.float32)
        return carry

    lax.fori_loop(0, ECH * (DD // 16), zb, None)
    for sblk in range(STRIPE // ECH):
        pltpu.sync_copy(rows_v, acc_sh.at[pl.ds(s * STRIPE + sblk * ECH, ECH)])
    plsc.subcore_barrier()

    def chunk(k, carry):
        g = k * SC_NW + wid

        @pl.when(g < NCHUNK)
        def _():
            base = g * ECH
            pltpu.sync_copy(edge_ref.at[0, pl.ds(base, ECH)], src_v)
            pltpu.sync_copy(edge_ref.at[1, pl.ds(base, ECH)], dst_v)
            pltpu.async_copy(hs_ref.at[src_v], rows_v, sem).wait()
            pltpu.sync_copy(rows_v, acc_sh.at[dst_v], add=True)

        return carry

    lax.fori_loop(0, KMAX, chunk, None)
    plsc.subcore_barrier()
    pltpu.sync_copy(acc_sh.at[pl.ds(s * STRIPE, STRIPE)],
                    acc_ref.at[c, pl.ds(s * STRIPE, STRIPE)])


def kernel(edge1, edge2, feat1, feat2, W_gcn, b_gcn, fc1_W, fc1_b, fc2_W,
           fc2_b):
    f1p = jnp.pad(feat1, ((0, NPAD - NN), (0, 0)))
    f2p = jnp.pad(feat2, ((0, NPAD - NN), (0, 0)))
    degp1 = jnp.pad(_sc_deg(edge1), ((0, 0), (0, NPAD - NN)))
    degp2 = jnp.pad(_sc_deg(edge2), ((0, 0), (0, NPAD - NN)))
    hs1 = _k1(f1p, W_gcn, degp1)
    hs2 = _k1(f2p, W_gcn, degp2)
    accp1 = _sc_scatter(hs1, edge1)
    accp2 = _sc_scatter(hs2, edge2)
    b2d = b_gcn.reshape(1, DD)
    b1d = fc1_b.reshape(1, DD)
    b2d2 = fc2_b.reshape(1, DD)
    a1p = _k5(accp1[0], accp1[1], hs1, degp1, b2d, fc1_W, b1d, fc2_W, b2d2)
    a2p = _k5(accp2[0], accp2[1], hs2, degp2, b2d, fc1_W, b1d, fc2_W, b2d2)
    r11, r12, d11, d12, r22, c12, d22 = _k6(a1p.astype(jnp.bfloat16),
                                            a2p.astype(jnp.bfloat16))
    loss = _k7(r11, r12, d11, d12, r22, c12, d22)
    return loss[0, 0]


# speedup vs baseline: 10.8358x; 1.2192x over previous
import functools

import jax
import jax.numpy as jnp
from jax import lax
from jax.experimental import pallas as pl
from jax.experimental.pallas import tpu as pltpu
from jax.experimental.pallas import tpu_sc as plsc

NN = 10000
DD = 128
EE = 160000
SC_NC = 2
SC_NS = 16
SC_NW = SC_NC * SC_NS
ECH = 128
NCHUNK = EE // ECH
KMAX = -(-NCHUNK // SC_NW)
NCHP = 1280
CPT = NCHP // SC_NS
ACC_ROWS = 10240
STRIPE = ACC_ROWS // SC_NS
NPAD = 10240
BI = 512
BJ = 512
NIB = NPAD // BI
NJB = NPAD // BJ
INV_TEMP = 2.0


def _dinv_block(degp_ref, i, B):
    deg = jnp.sum(degp_ref[:, pl.ds(i * B, B)], axis=0) + 1.0
    return lax.rsqrt(deg)[:, None]


def _k1_body(x_ref, w_ref, degp_ref, hs_ref):
    i = pl.program_id(0)
    dinv = _dinv_block(degp_ref, i, x_ref.shape[0])
    h = jnp.dot(x_ref[...], w_ref[...], preferred_element_type=jnp.float32)
    hs_ref[...] = h * dinv


def _k1(x, w, degp):
    B = 2048
    grid = (NPAD // B,)
    return pl.pallas_call(
        _k1_body,
        grid=grid,
        in_specs=[
            pl.BlockSpec((B, DD), lambda i: (i, 0)),
            pl.BlockSpec((DD, DD), lambda i: (0, 0)),
            pl.BlockSpec((SC_NS, NPAD), lambda i: (0, 0)),
        ],
        out_specs=pl.BlockSpec((B, DD), lambda i: (i, 0)),
        out_shape=jax.ShapeDtypeStruct((NPAD, DD), jnp.float32),
    )(x, w, degp)


def _k5_body(acc_ref, hs_ref, degp_ref, b_ref, w1_ref, b1_ref,
             w2_ref, b2_ref, a_ref):
    i = pl.program_id(0)
    dinv = _dinv_block(degp_ref, i, acc_ref.shape[0])
    g = dinv * (acc_ref[...] + hs_ref[...]) + b_ref[...]
    t = jnp.dot(g, w1_ref[...], preferred_element_type=jnp.float32) + b1_ref[...]
    z = jnp.where(t > 0.0, t, jnp.exp(jnp.minimum(t, 0.0)) - 1.0)
    z2 = jnp.dot(z, w2_ref[...], preferred_element_type=jnp.float32) + b2_ref[...]
    nrm = jnp.sqrt(jnp.sum(z2 * z2, axis=1, keepdims=True))
    a_ref[...] = z2 / jnp.maximum(nrm, 1e-12)


def _k5(acc, hs, degp, b, w1, b1, w2, b2):
    B = 2048
    grid = (NPAD // B,)
    row = lambda i: (i, 0)
    full = lambda i: (0, 0)
    return pl.pallas_call(
        _k5_body,
        grid=grid,
        in_specs=[
            pl.BlockSpec((B, DD), row),
            pl.BlockSpec((B, DD), row),
            pl.BlockSpec((SC_NS, NPAD), full),
            pl.BlockSpec((1, DD), full),
            pl.BlockSpec((DD, DD), full),
            pl.BlockSpec((1, DD), full),
            pl.BlockSpec((DD, DD), full),
            pl.BlockSpec((1, DD), full),
        ],
        out_specs=pl.BlockSpec((B, DD), row),
        out_shape=jax.ShapeDtypeStruct((NPAD, DD), jnp.float32),
    )(acc, hs, degp, b, w1, b1, w2, b2)


def _dott(x, y):
    return lax.dot_general(x, y, (((1,), (1,)), ((), ())),
                           preferred_element_type=jnp.float32)


def _k6_body(aI_ref, bI_ref, aJ_ref, bJ_ref,
             r11_ref, r12_ref, d11_ref, d12_ref,
             r22_ref, c12_ref, d22_ref):
    i = pl.program_id(0)
    j = pl.program_id(1)
    aI = aI_ref[...]
    bI = bI_ref[...]
    aJ = aJ_ref[...]
    bJ = bJ_ref[...]
    s11 = _dott(aI, aJ)
    s22t = _dott(bJ, bI)
    s12 = _dott(aI, bJ)
    e11 = jnp.exp(s11 * INV_TEMP)
    e22t = jnp.exp(s22t * INV_TEMP)
    e12 = jnp.exp(s12 * INV_TEMP)
    jm1 = ((lax.broadcasted_iota(jnp.int32, (1, BJ), 1) + j * BJ) < NN
           ).astype(jnp.float32)
    im1 = ((lax.broadcasted_iota(jnp.int32, (1, BI), 1) + i * BI) < NN
           ).astype(jnp.float32)

    @pl.when(j == 0)
    def _():
        r11_ref[...] = jnp.zeros_like(r11_ref)
        r12_ref[...] = jnp.zeros_like(r12_ref)
        d11_ref[...] = jnp.zeros_like(d11_ref)
        d12_ref[...] = jnp.zeros_like(d12_ref)

    @pl.when(jnp.logical_and(i == 0, j == 0))
    def _():
        r22_ref[...] = jnp.zeros_like(r22_ref)
        c12_ref[...] = jnp.zeros_like(c12_ref)
        d22_ref[...] = jnp.zeros_like(d22_ref)

    r11_ref[...] += _dott(e11, jm1)
    r12_ref[...] += _dott(e12, jm1)
    r22_ref[pl.ds(i, 1), :] += jnp.dot(jm1, e22t,
                                       preferred_element_type=jnp.float32)
    c12_ref[pl.ds(j, 1), :] += jnp.dot(im1, e12,
                                       preferred_element_type=jnp.float32)

    @pl.when(i == j)
    def _():
        eye = (lax.broadcasted_iota(jnp.int32, (BI, BJ), 0)
               == lax.broadcasted_iota(jnp.int32, (BI, BJ), 1)
               ).astype(jnp.float32)
        d11_ref[...] += _dott(e11 * eye, jm1)
        d12_ref[...] += _dott(s12 * eye, jm1)
        d22_ref[pl.ds(i, 1), :] += jnp.dot(jm1, e22t * eye,
                                           preferred_element_type=jnp.float32)


def _k6(a1p, a2p):
    colspec = pl.BlockSpec((BI, 1), lambda i, j: (i, 0))
    col = jax.ShapeDtypeStruct((NPAD, 1), jnp.float32)
    lanespec = pl.BlockSpec((NIB, BI), lambda i, j: (0, 0))
    lane = jax.ShapeDtypeStruct((NIB, BI), jnp.float32)
    return pl.pallas_call(
        _k6_body,
        grid=(NIB, NJB),
        in_specs=[
            pl.BlockSpec((BI, DD), lambda i, j: (i, 0)),
            pl.BlockSpec((BI, DD), lambda i, j: (i, 0)),
            pl.BlockSpec((BJ, DD), lambda i, j: (j, 0)),
            pl.BlockSpec((BJ, DD), lambda i, j: (j, 0)),
        ],
        out_specs=[colspec, colspec, colspec, colspec,
                   lanespec, lanespec, lanespec],
        out_shape=[col, col, col, col, lane, lane, lane],
    )(a1p, a1p, a2p, a2p)


def _k7_body(r11_ref, r12_ref, d11_ref, d12_ref, r22_ref, c12_ref, d22_ref,
             out_ref):
    k = pl.program_id(0)
    imc = ((lax.broadcasted_iota(jnp.int32, (BI, 1), 0) + k * BI) < NN
           ).astype(jnp.float32)
    x1 = r11_ref[...] + r12_ref[...] - d11_ref[...]
    p1 = jnp.sum((0.5 * jnp.log(x1) - INV_TEMP * d12_ref[...]) * imc)
    iml = ((lax.broadcasted_iota(jnp.int32, (1, BI), 1) + k * BI) < NN
           ).astype(jnp.float32)
    rk = pl.ds(k, 1)
    x2 = r22_ref[rk, :] + c12_ref[rk, :] - d22_ref[rk, :]
    p2 = jnp.sum(0.5 * jnp.log(x2) * iml)

    @pl.when(k == 0)
    def _():
        out_ref[0, 0] = 0.0

    out_ref[0, 0] += (p1 + p2) * (1.0 / NN)


def _k7(r11, r12, d11, d12, r22, c12, d22):
    colspec = pl.BlockSpec((BI, 1), lambda k: (k, 0))
    lanespec = pl.BlockSpec((NIB, BI), lambda k: (0, 0))
    return pl.pallas_call(
        _k7_body,
        grid=(NIB,),
        in_specs=[colspec, colspec, colspec, colspec,
                  lanespec, lanespec, lanespec],
        out_specs=pl.BlockSpec(memory_space=pltpu.SMEM),
        out_shape=jax.ShapeDtypeStruct((1, 1), jnp.float32),
    )(r11, r12, d11, d12, r22, c12, d22)


@functools.lru_cache(maxsize=None)
def _sc_mesh():
    return plsc.VectorSubcoreMesh(core_axis_name="c", subcore_axis_name="s",
                                  num_cores=SC_NC, num_subcores=SC_NS)


def _sc_deg(edge_st):
    return pl.kernel(
        _sc_deg_body,
        out_type=jax.ShapeDtypeStruct((SC_NC, SC_NS, NN), jnp.float32),
        mesh=_sc_mesh(),
        scratch_types=[
            pltpu.VMEM((CPT, ECH), jnp.int32),
            pltpu.VMEM((ACC_ROWS,), jnp.float32),
        ],
        compiler_params=pltpu.CompilerParams(needs_layout_passes=False,
                                             use_tc_tiling_on_sc=False),
    )(edge_st)


def _sc_deg_body(edge_ref, deg_ref, idx_v, degloc):
    c = lax.axis_index("c")
    s = lax.axis_index("s")
    base = s * CPT

    def zb(k, carry):
        degloc[pl.ds(k * 16, 16)] = jnp.zeros((16,), jnp.float32)
        return carry

    lax.fori_loop(0, ACC_ROWS // 16, zb, None)
    pltpu.sync_copy(edge_ref.at[c, 1, pl.ds(base, CPT)], idx_v)
    ones = jnp.ones((16,), jnp.float32)

    def chunk(kl, carry):
        g = base + kl

        @pl.when(g < NCHUNK)
        def _():
            for t in range(ECH // 16):
                idx = idx_v[kl, pl.ds(t * 16, 16)]
                plsc.addupdate_scatter(degloc, [idx], ones)

        return carry

    lax.fori_loop(0, CPT, chunk, None)
    pltpu.sync_copy(degloc.at[pl.ds(0, NN)], deg_ref.at[c, s])


def _sc_scatter(hs_st, edge_st):
    return pl.kernel(
        _sc_scatter_body,
        out_type=jax.ShapeDtypeStruct((SC_NC, ACC_ROWS, DD), jnp.float32),
        mesh=_sc_mesh(),
        scratch_types=[
            pltpu.VMEM((CPT // 2, ECH), jnp.int32),
            pltpu.VMEM((CPT // 2, ECH), jnp.int32),
            pltpu.VMEM((ECH, DD), jnp.float32),
            pltpu.VMEM((ECH, DD), jnp.float32),
            pltpu.VMEM_SHARED((ACC_ROWS, DD), jnp.float32),
            pltpu.SemaphoreType.DMA,
            pltpu.SemaphoreType.DMA,
        ],
        compiler_params=pltpu.CompilerParams(needs_layout_passes=False,
                                             use_tc_tiling_on_sc=False),
    )(hs_st, edge_st)


def _sc_scatter_body(hs_ref, edge_ref, acc_ref, src_v, dst_v, rows0, rows1,
                     acc_sh, sem0, sem1):
    c = lax.axis_index("c")
    s = lax.axis_index("s")
    base = s * CPT
    hs_c = hs_ref.at[c]

    def zb(k, carry):
        r = k // (DD // 16)
        t = k % (DD // 16)
        rows0[r, pl.ds(t * 16, 16)] = jnp.zeros((16,), jnp.float32)
        return carry

    lax.fori_loop(0, ECH * (DD // 16), zb, None)
    for sblk in range(STRIPE // ECH):
        pltpu.sync_copy(rows0, acc_sh.at[pl.ds(s * STRIPE + sblk * ECH, ECH)])
    plsc.subcore_barrier()

    bufs = (rows0, rows1)
    sems = (sem0, sem1)
    HC = CPT // 2

    for h in range(2):
        hbase = base + h * HC
        pltpu.sync_copy(edge_ref.at[c, 0, pl.ds(hbase, HC)], src_v)
        pltpu.sync_copy(edge_ref.at[c, 1, pl.ds(hbase, HC)], dst_v)

        @pl.when(hbase < NCHUNK)
        def _():
            pltpu.async_copy(hs_c.at[src_v.at[0]], rows0, sem0)

        def pair(k2, carry):
            for ph in range(2):
                kl = 2 * k2 + ph
                g = hbase + kl
                buf = bufs[ph]
                sem = sems[ph]
                obuf = bufs[1 - ph]
                osem = sems[1 - ph]

                @pl.when(g < NCHUNK)
                def _():
                    pltpu.make_async_copy(hs_c.at[src_v.at[kl]], buf,
                                          sem).wait()

                @pl.when(jnp.logical_and(g + 1 < NCHUNK, kl + 1 < HC))
                def _():
                    pltpu.async_copy(hs_c.at[src_v.at[kl + 1]], obuf, osem)

                @pl.when(g < NCHUNK)
                def _():
                    pltpu.sync_copy(buf, acc_sh.at[dst_v.at[kl]], add=True)

            return carry

        lax.fori_loop(0, HC // 2, pair, None)
    plsc.subcore_barrier()
    pltpu.sync_copy(acc_sh.at[pl.ds(s * STRIPE, STRIPE)],
                    acc_ref.at[c, pl.ds(s * STRIPE, STRIPE)])


def kernel(edge1, edge2, feat1, feat2, W_gcn, b_gcn, fc1_W, fc1_b, fc2_W,
           fc2_b):
    f1p = jnp.pad(feat1, ((0, NPAD - NN), (0, 0)))
    f2p = jnp.pad(feat2, ((0, NPAD - NN), (0, 0)))
    e1r = jnp.pad(edge1.reshape(2, NCHUNK, ECH),
                  ((0, 0), (0, NCHP - NCHUNK), (0, 0)))
    e2r = jnp.pad(edge2.reshape(2, NCHUNK, ECH),
                  ((0, 0), (0, NCHP - NCHUNK), (0, 0)))
    edge_st = jnp.stack([e1r, e2r])
    degp = _sc_deg(edge_st)
    degp1 = jnp.pad(degp[0], ((0, 0), (0, NPAD - NN)))
    degp2 = jnp.pad(degp[1], ((0, 0), (0, NPAD - NN)))
    hs1 = _k1(f1p, W_gcn, degp1)
    hs2 = _k1(f2p, W_gcn, degp2)
    accp = _sc_scatter(jnp.stack([hs1, hs2]), edge_st)
    b2d = b_gcn.reshape(1, DD)
    b1d = fc1_b.reshape(1, DD)
    b2d2 = fc2_b.reshape(1, DD)
    a1p = _k5(accp[0], hs1, degp1, b2d, fc1_W, b1d, fc2_W, b2d2)
    a2p = _k5(accp[1], hs2, degp2, b2d, fc1_W, b1d, fc2_W, b2d2)
    r11, r12, d11, d12, r22, c12, d22 = _k6(a1p.astype(jnp.bfloat16),
                                            a2p.astype(jnp.bfloat16))
    loss = _k7(r11, r12, d11, d12, r22, c12, d22)
    return loss[0, 0]


# speedup vs baseline: 13.0165x; 1.2013x over previous
import functools

import jax
import jax.numpy as jnp
from jax import lax
from jax.experimental import pallas as pl
from jax.experimental.pallas import tpu as pltpu
from jax.experimental.pallas import tpu_sc as plsc

NN = 10000
DD = 128
EE = 160000
SC_NC = 2
SC_NS = 16
SC_NW = SC_NC * SC_NS
ECH = 128
NCHUNK = EE // ECH
KMAX = -(-NCHUNK // SC_NW)
NCHP = 1280
CPT = NCHP // SC_NS
ACC_ROWS = 10240
STRIPE = ACC_ROWS // SC_NS
NPAD = 10240
BI = 1024
BJ = 1024
NIB = NPAD // BI
NJB = NPAD // BJ
INV_TEMP = 2.0


def _dinv_block(degp_ref, i, B):
    deg = jnp.sum(degp_ref[:, pl.ds(i * B, B)], axis=0) + 1.0
    return lax.rsqrt(deg)[:, None]


def _k1_body(x_ref, w_ref, degp_ref, hs_ref):
    i = pl.program_id(0)
    dinv = _dinv_block(degp_ref, i, x_ref.shape[0])
    h = jnp.dot(x_ref[...], w_ref[...], preferred_element_type=jnp.float32)
    hs_ref[...] = h * dinv


def _k1(x, w, degp):
    B = 2048
    grid = (NPAD // B,)
    return pl.pallas_call(
        _k1_body,
        grid=grid,
        in_specs=[
            pl.BlockSpec((B, DD), lambda i: (i, 0)),
            pl.BlockSpec((DD, DD), lambda i: (0, 0)),
            pl.BlockSpec((SC_NS, NPAD), lambda i: (0, 0)),
        ],
        out_specs=pl.BlockSpec((B, DD), lambda i: (i, 0)),
        out_shape=jax.ShapeDtypeStruct((NPAD, DD), jnp.float32),
    )(x, w, degp)


def _k5_body(acc_ref, hs_ref, degp_ref, b_ref, w1_ref, b1_ref,
             w2_ref, b2_ref, a_ref):
    i = pl.program_id(0)
    dinv = _dinv_block(degp_ref, i, acc_ref.shape[0])
    g = dinv * (acc_ref[...] + hs_ref[...]) + b_ref[...]
    t = jnp.dot(g, w1_ref[...], preferred_element_type=jnp.float32) + b1_ref[...]
    z = jnp.where(t > 0.0, t, jnp.exp(jnp.minimum(t, 0.0)) - 1.0)
    z2 = jnp.dot(z, w2_ref[...], preferred_element_type=jnp.float32) + b2_ref[...]
    nrm = jnp.sqrt(jnp.sum(z2 * z2, axis=1, keepdims=True))
    a_ref[...] = z2 / jnp.maximum(nrm, 1e-12)


def _k5(acc, hs, degp, b, w1, b1, w2, b2):
    B = 2048
    grid = (NPAD // B,)
    row = lambda i: (i, 0)
    full = lambda i: (0, 0)
    return pl.pallas_call(
        _k5_body,
        grid=grid,
        in_specs=[
            pl.BlockSpec((B, DD), row),
            pl.BlockSpec((B, DD), row),
            pl.BlockSpec((SC_NS, NPAD), full),
            pl.BlockSpec((1, DD), full),
            pl.BlockSpec((DD, DD), full),
            pl.BlockSpec((1, DD), full),
            pl.BlockSpec((DD, DD), full),
            pl.BlockSpec((1, DD), full),
        ],
        out_specs=pl.BlockSpec((B, DD), row),
        out_shape=jax.ShapeDtypeStruct((NPAD, DD), jnp.float32),
    )(acc, hs, degp, b, w1, b1, w2, b2)


def _dott(x, y):
    return lax.dot_general(x, y, (((1,), (1,)), ((), ())),
                           preferred_element_type=jnp.float32)


def _k6_body(aI_ref, bI_ref, aJ_ref, bJ_ref,
             r11_ref, r12_ref, d11_ref, d12_ref,
             r22_ref, c12_ref, d22_ref):
    i = pl.program_id(0)
    j = pl.program_id(1)
    aI = aI_ref[...]
    bI = bI_ref[...]
    aJ = aJ_ref[...]
    bJ = bJ_ref[...]
    s11 = _dott(aI, aJ)
    s22t = _dott(bJ, bI)
    s12 = _dott(aI, bJ)
    e11 = jnp.exp(s11)
    e22t = jnp.exp(s22t)
    e12 = jnp.exp(s12)
    jm1 = ((lax.broadcasted_iota(jnp.int32, (1, BJ), 1) + j * BJ) < NN
           ).astype(jnp.float32)
    im1 = ((lax.broadcasted_iota(jnp.int32, (1, BI), 1) + i * BI) < NN
           ).astype(jnp.float32)

    @pl.when(j == 0)
    def _():
        r11_ref[...] = jnp.zeros_like(r11_ref)
        r12_ref[...] = jnp.zeros_like(r12_ref)
        d11_ref[...] = jnp.zeros_like(d11_ref)
        d12_ref[...] = jnp.zeros_like(d12_ref)

    @pl.when(jnp.logical_and(i == 0, j == 0))
    def _():
        r22_ref[...] = jnp.zeros_like(r22_ref)
        c12_ref[...] = jnp.zeros_like(c12_ref)
        d22_ref[...] = jnp.zeros_like(d22_ref)

    r11_ref[...] += _dott(e11, jm1)
    r12_ref[...] += _dott(e12, jm1)
    r22_ref[pl.ds(i, 1), :] += jnp.dot(jm1, e22t,
                                       preferred_element_type=jnp.float32)
    c12_ref[pl.ds(j, 1), :] += jnp.dot(im1, e12,
                                       preferred_element_type=jnp.float32)

    @pl.when(i == j)
    def _():
        eye = (lax.broadcasted_iota(jnp.int32, (BI, BJ), 0)
               == lax.broadcasted_iota(jnp.int32, (BI, BJ), 1)
               ).astype(jnp.float32)
        d11_ref[...] += _dott(e11 * eye, jm1)
        d12_ref[...] += _dott(s12 * eye, jm1)
        d22_ref[pl.ds(i, 1), :] += jnp.dot(jm1, e22t * eye,
                                           preferred_element_type=jnp.float32)


def _k6(a1p, a2p, a1s, a2s):
    colspec = pl.BlockSpec((BI, 1), lambda i, j: (i, 0))
    col = jax.ShapeDtypeStruct((NPAD, 1), jnp.float32)
    lanespec = pl.BlockSpec((NIB, BI), lambda i, j: (0, 0))
    lane = jax.ShapeDtypeStruct((NIB, BI), jnp.float32)
    return pl.pallas_call(
        _k6_body,
        grid=(NIB, NJB),
        in_specs=[
            pl.BlockSpec((BI, DD), lambda i, j: (i, 0)),
            pl.BlockSpec((BI, DD), lambda i, j: (i, 0)),
            pl.BlockSpec((BJ, DD), lambda i, j: (j, 0)),
            pl.BlockSpec((BJ, DD), lambda i, j: (j, 0)),
        ],
        out_specs=[colspec, colspec, colspec, colspec,
                   lanespec, lanespec, lanespec],
        out_shape=[col, col, col, col, lane, lane, lane],
    )(a1p, a2p, a1s, a2s)


def _k7_body(r11_ref, r12_ref, d11_ref, d12_ref, r22_ref, c12_ref, d22_ref,
             out_ref):
    k = pl.program_id(0)
    imc = ((lax.broadcasted_iota(jnp.int32, (BI, 1), 0) + k * BI) < NN
           ).astype(jnp.float32)
    x1 = r11_ref[...] + r12_ref[...] - d11_ref[...]
    p1 = jnp.sum((0.5 * jnp.log(x1) - d12_ref[...]) * imc)
    iml = ((lax.broadcasted_iota(jnp.int32, (1, BI), 1) + k * BI) < NN
           ).astype(jnp.float32)
    rk = pl.ds(k, 1)
    x2 = r22_ref[rk, :] + c12_ref[rk, :] - d22_ref[rk, :]
    p2 = jnp.sum(0.5 * jnp.log(x2) * iml)

    @pl.when(k == 0)
    def _():
        out_ref[0, 0] = 0.0

    out_ref[0, 0] += (p1 + p2) * (1.0 / NN)


def _k7(r11, r12, d11, d12, r22, c12, d22):
    colspec = pl.BlockSpec((BI, 1), lambda k: (k, 0))
    lanespec = pl.BlockSpec((NIB, BI), lambda k: (0, 0))
    return pl.pallas_call(
        _k7_body,
        grid=(NIB,),
        in_specs=[colspec, colspec, colspec, colspec,
                  lanespec, lanespec, lanespec],
        out_specs=pl.BlockSpec(memory_space=pltpu.SMEM),
        out_shape=jax.ShapeDtypeStruct((1, 1), jnp.float32),
    )(r11, r12, d11, d12, r22, c12, d22)


@functools.lru_cache(maxsize=None)
def _sc_mesh():
    return plsc.VectorSubcoreMesh(core_axis_name="c", subcore_axis_name="s",
                                  num_cores=SC_NC, num_subcores=SC_NS)


def _sc_deg(edge_st):
    return pl.kernel(
        _sc_deg_body,
        out_type=jax.ShapeDtypeStruct((SC_NC, SC_NS, NN), jnp.float32),
        mesh=_sc_mesh(),
        scratch_types=[
            pltpu.VMEM((CPT, ECH), jnp.int32),
            pltpu.VMEM((ACC_ROWS,), jnp.float32),
        ],
        compiler_params=pltpu.CompilerParams(needs_layout_passes=False,
                                             use_tc_tiling_on_sc=False),
    )(edge_st)


def _sc_deg_body(edge_ref, deg_ref, idx_v, degloc):
    c = lax.axis_index("c")
    s = lax.axis_index("s")
    base = s * CPT

    def zb(k, carry):
        degloc[pl.ds(k * 16, 16)] = jnp.zeros((16,), jnp.float32)
        return carry

    lax.fori_loop(0, ACC_ROWS // 16, zb, None)
    pltpu.sync_copy(edge_ref.at[c, 1, pl.ds(base, CPT)], idx_v)
    ones = jnp.ones((16,), jnp.float32)

    def chunk(kl, carry):
        g = base + kl

        @pl.when(g < NCHUNK)
        def _():
            for t in range(ECH // 16):
                idx = idx_v[kl, pl.ds(t * 16, 16)]
                plsc.addupdate_scatter(degloc, [idx], ones)

        return carry

    lax.fori_loop(0, CPT, chunk, None)
    pltpu.sync_copy(degloc.at[pl.ds(0, NN)], deg_ref.at[c, s])


def _sc_scatter(hs_st, edge_st):
    return pl.kernel(
        _sc_scatter_body,
        out_type=jax.ShapeDtypeStruct((SC_NC, ACC_ROWS, DD), jnp.float32),
        mesh=_sc_mesh(),
        scratch_types=[
            pltpu.VMEM((CPT // 2, ECH), jnp.int32),
            pltpu.VMEM((CPT // 2, ECH), jnp.int32),
            pltpu.VMEM((ECH, DD), jnp.float32),
            pltpu.VMEM((ECH, DD), jnp.float32),
            pltpu.VMEM_SHARED((ACC_ROWS, DD), jnp.float32),
            pltpu.SemaphoreType.DMA,
            pltpu.SemaphoreType.DMA,
        ],
        compiler_params=pltpu.CompilerParams(needs_layout_passes=False,
                                             use_tc_tiling_on_sc=False),
    )(hs_st, edge_st)


def _sc_scatter_body(hs_ref, edge_ref, acc_ref, src_v, dst_v, rows0, rows1,
                     acc_sh, sem0, sem1):
    c = lax.axis_index("c")
    s = lax.axis_index("s")
    base = s * CPT
    hs_c = hs_ref.at[c]

    def zb(k, carry):
        r = k // (DD // 16)
        t = k % (DD // 16)
        rows0[r, pl.ds(t * 16, 16)] = jnp.zeros((16,), jnp.float32)
        return carry

    lax.fori_loop(0, ECH * (DD // 16), zb, None)
    for sblk in range(STRIPE // ECH):
        pltpu.sync_copy(rows0, acc_sh.at[pl.ds(s * STRIPE + sblk * ECH, ECH)])
    plsc.subcore_barrier()

    bufs = (rows0, rows1)
    sems = (sem0, sem1)
    HC = CPT // 2

    for h in range(2):
        hbase = base + h * HC
        pltpu.sync_copy(edge_ref.at[c, 0, pl.ds(hbase, HC)], src_v)
        pltpu.sync_copy(edge_ref.at[c, 1, pl.ds(hbase, HC)], dst_v)

        @pl.when(hbase < NCHUNK)
        def _():
            pltpu.async_copy(hs_c.at[src_v.at[0]], rows0, sem0)

        def pair(k2, carry):
            for ph in range(2):
                kl = 2 * k2 + ph
                g = hbase + kl
                buf = bufs[ph]
                sem = sems[ph]
                obuf = bufs[1 - ph]
                osem = sems[1 - ph]

                @pl.when(g < NCHUNK)
                def _():
                    pltpu.make_async_copy(hs_c.at[src_v.at[kl]], buf,
                                          sem).wait()

                @pl.when(jnp.logical_and(g + 1 < NCHUNK, kl + 1 < HC))
                def _():
                    pltpu.async_copy(hs_c.at[src_v.at[kl + 1]], obuf, osem)

                @pl.when(g < NCHUNK)
                def _():
                    pltpu.sync_copy(buf, acc_sh.at[dst_v.at[kl]], add=True)

            return carry

        lax.fori_loop(0, HC // 2, pair, None)
    plsc.subcore_barrier()
    pltpu.sync_copy(acc_sh.at[pl.ds(s * STRIPE, STRIPE)],
                    acc_ref.at[c, pl.ds(s * STRIPE, STRIPE)])


def kernel(edge1, edge2, feat1, feat2, W_gcn, b_gcn, fc1_W, fc1_b, fc2_W,
           fc2_b):
    f1p = jnp.pad(feat1, ((0, NPAD - NN), (0, 0)))
    f2p = jnp.pad(feat2, ((0, NPAD - NN), (0, 0)))
    e1r = jnp.pad(edge1.reshape(2, NCHUNK, ECH),
                  ((0, 0), (0, NCHP - NCHUNK), (0, 0)))
    e2r = jnp.pad(edge2.reshape(2, NCHUNK, ECH),
                  ((0, 0), (0, NCHP - NCHUNK), (0, 0)))
    edge_st = jnp.stack([e1r, e2r])
    degp = _sc_deg(edge_st)
    degp1 = jnp.pad(degp[0], ((0, 0), (0, NPAD - NN)))
    degp2 = jnp.pad(degp[1], ((0, 0), (0, NPAD - NN)))
    hs1 = _k1(f1p, W_gcn, degp1)
    hs2 = _k1(f2p, W_gcn, degp2)
    accp = _sc_scatter(jnp.stack([hs1, hs2]), edge_st)
    b2d = b_gcn.reshape(1, DD)
    b1d = fc1_b.reshape(1, DD)
    b2d2 = fc2_b.reshape(1, DD)
    a1p = _k5(accp[0], hs1, degp1, b2d, fc1_W, b1d, fc2_W, b2d2)
    a2p = _k5(accp[1], hs2, degp2, b2d, fc1_W, b1d, fc2_W, b2d2)
    r11, r12, d11, d12, r22, c12, d22 = _k6(
        a1p.astype(jnp.bfloat16), a2p.astype(jnp.bfloat16),
        (a1p * INV_TEMP).astype(jnp.bfloat16),
        (a2p * INV_TEMP).astype(jnp.bfloat16))
    loss = _k7(r11, r12, d11, d12, r22, c12, d22)
    return loss[0, 0]
